# baseline (speedup 1.0000x reference)
import jax
import jax.numpy as jnp
from jax.experimental import pallas as pl
from jax.experimental.pallas import tpu as pltpu
from jax.experimental.pallas import tpu_sc as plsc

B = 8
NPTS = 2048
NTOT = B * NPTS
E = 16
GEO = 7
SHAPE = 128
T = 256
NT = NTOT // T + E
BUFROWS = NT * T
COLS = 256
LAT = 1024
DEC_OUT = 14336
F32 = jnp.float32
BF16 = jnp.bfloat16


def _routing_body(cats_ref, dest_ref, te_ref, nv_ref):
    c = cats_ref[...]
    r_iota = jax.lax.broadcasted_iota(jnp.int32, (128, 128), 0)
    c_iota = jax.lax.broadcasted_iota(jnp.int32, (128, 128), 1)
    u_mat = (r_iota <= c_iota).astype(BF16)
    l_mat = (c_iota < r_iota).astype(BF16)

    counts, starts, ends = [], [], []
    run = jnp.float32(0.0)
    for e in range(E):
        cnt = jnp.sum((c == e).astype(F32))
        counts.append(cnt)
        starts.append(run)
        run = run + jnp.ceil(cnt / T) * T
        ends.append(run)

    dest = jnp.zeros((128, 128), F32)
    for e in range(E):
        m_f = (c == e).astype(F32)
        lane_pre = jax.lax.dot(m_f.astype(BF16), u_mat,
                               preferred_element_type=F32)
        rowsum = lane_pre[:, 127:128]
        rowpre = jax.lax.dot(l_mat, rowsum.astype(BF16),
                             preferred_element_type=F32)
        cum = lane_pre + rowpre
        dest = dest + m_f * (starts[e] + cum - 1.0)
    dest_ref[...] = dest.astype(jnp.int32)

    tv = jax.lax.broadcasted_iota(jnp.int32, (1, NT), 1).astype(F32) * T
    te = jnp.zeros((1, NT), F32)
    nv = jnp.zeros((1, NT), F32)
    for e in range(E):
        te = te + (tv >= ends[e]).astype(F32)
        inb = ((tv >= starts[e]) & (tv < ends[e])).astype(F32)
        nv = nv + inb * jnp.clip(counts[e] - (tv - starts[e]), 0.0, float(T))
    te_ref[...] = jnp.minimum(te, float(E - 1)).astype(jnp.int32)
    nv_ref[...] = nv.astype(jnp.int32)


def _routing(cats32):
    return pl.pallas_call(
        _routing_body,
        out_shape=[
            jax.ShapeDtypeStruct((128, 128), jnp.int32),
            jax.ShapeDtypeStruct((1, NT), jnp.int32),
            jax.ShapeDtypeStruct((1, NT), jnp.int32),
        ],
    )(cats32)


def _transpose_body(x_ref, xt_ref):
    b = pl.program_id(0)
    xb = x_ref[0]
    tr = xb.T
    btag = jnp.full((512, 1), 1.0, F32) * (b.astype(F32) + 1.0)
    out = jnp.concatenate(
        [tr[:, :GEO], btag, tr[:, GEO:], jnp.zeros((512, COLS - 136), F32)],
        axis=1)
    xt_ref[...] = out.astype(BF16)


def _transpose(x):
    return pl.pallas_call(
        _transpose_body,
        grid=(B, NPTS // 512),
        in_specs=[pl.BlockSpec((1, 135, 512), lambda b, j: (b, 0, j))],
        out_specs=pl.BlockSpec((512, COLS), lambda b, j: (b * (NPTS // 512) + j, 0)),
        out_shape=jax.ShapeDtypeStruct((NTOT, COLS), BF16),
    )(x)


def _sc_scatter(xt_bits, dest_row):
    mesh = plsc.VectorSubcoreMesh(core_axis_name="c", subcore_axis_name="s")

    @pl.kernel(out_type=jax.ShapeDtypeStruct((BUFROWS, COLS // 2), jnp.int32),
               mesh=mesh)
    def k(x_hbm, i_hbm, o_hbm):
        def body(x_vmem, i_vmem):
            pltpu.sync_copy(x_vmem, o_hbm.at[i_vmem.at[0]])

        pltpu.emit_pipeline(
            body,
            grid=(NTOT // 128,),
            in_specs=[
                pl.BlockSpec((128, COLS // 2), lambda i: (i, 0)),
                pl.BlockSpec((1, 128), lambda i: (0, i)),
            ],
            out_specs=[],
            core_axis_name=("c", "s"),
            dimension_semantics=(pltpu.PARALLEL,),
        )(x_hbm, i_hbm)

    return k(xt_bits, dest_row)


def _main_body(te_ref, nv_ref, buf_ref, w1_ref, b1_ref, w2_ref, b2_ref,
               ew1g_ref, ew1c_ref, eb1_ref, ew2_ref, eb2_ref, ew3_ref,
               eb3_ref, lat_ref):
    t = pl.program_id(0)

    @pl.when(t == 0)
    def _():
        lat_ref[...] = jnp.full((B, LAT), -jnp.inf, F32)

    buf = buf_ref[...]
    codes = buf[:, 8:136]
    h = jax.lax.dot(codes, w1_ref[0], preferred_element_type=F32) + b1_ref[0]
    h = jnp.maximum(h, 0.0).astype(BF16)
    sel = jax.lax.dot(h, w2_ref[0], preferred_element_type=F32) + b2_ref[0]
    geo8 = buf[:, 0:8]
    t1 = (jax.lax.dot(geo8, ew1g_ref[...], preferred_element_type=F32)
          + jax.lax.dot(sel.astype(BF16), ew1c_ref[...], preferred_element_type=F32)
          + eb1_ref[...])
    t1 = jnp.maximum(t1, 0.0).astype(BF16)
    t2 = jax.lax.dot(t1, ew2_ref[...], preferred_element_type=F32) + eb2_ref[...]
    t2 = jnp.maximum(t2, 0.0).astype(BF16)
    t3 = jax.lax.dot(t2, ew3_ref[...], preferred_element_type=F32)

    nv = nv_ref[t]
    row = jax.lax.broadcasted_iota(jnp.int32, (T, 1), 0)
    validrow = row < nv
    btag = buf[:, 7:8].astype(F32)
    neg = jnp.float32(-jnp.inf)
    for b in range(B):
        mask = validrow & (btag == jnp.float32(b + 1))
        cand = jnp.max(jnp.where(mask, t3, neg), axis=0, keepdims=True)
        lat_ref[b:b + 1, :] = jnp.maximum(lat_ref[b:b + 1, :], cand)

    @pl.when(t == NT - 1)
    def _():
        lat_ref[...] = lat_ref[...] + eb3_ref[...]


def _main(te, nv, buf, w1, b1, w2, b2, ew1g, ew1c, eb1, ew2, eb2, ew3, eb3):
    grid_spec = pltpu.PrefetchScalarGridSpec(
        num_scalar_prefetch=2,
        grid=(NT,),
        in_specs=[
            pl.BlockSpec((T, COLS), lambda t, te, nv: (t, 0)),
            pl.BlockSpec((1, SHAPE, 256), lambda t, te, nv: (te[t], 0, 0)),
            pl.BlockSpec((1, 1, 256), lambda t, te, nv: (te[t], 0, 0)),
            pl.BlockSpec((1, 256, SHAPE), lambda t, te, nv: (te[t], 0, 0)),
            pl.BlockSpec((1, 1, SHAPE), lambda t, te, nv: (te[t], 0, 0)),
            pl.BlockSpec((8, 256), lambda t, te, nv: (0, 0)),
            pl.BlockSpec((SHAPE, 256), lambda t, te, nv: (0, 0)),
            pl.BlockSpec((1, 256), lambda t, te, nv: (0, 0)),
            pl.BlockSpec((256, 512), lambda t, te, nv: (0, 0)),
            pl.BlockSpec((1, 512), lambda t, te, nv: (0, 0)),
            pl.BlockSpec((512, LAT), lambda t, te, nv: (0, 0)),
            pl.BlockSpec((1, LAT), lambda t, te, nv: (0, 0)),
        ],
        out_specs=pl.BlockSpec((B, LAT), lambda t, te, nv: (0, 0)),
    )
    return pl.pallas_call(
        _main_body,
        grid_spec=grid_spec,
        out_shape=jax.ShapeDtypeStruct((B, LAT), F32),
    )(te, nv, buf, w1, b1, w2, b2, ew1g, ew1c, eb1, ew2, eb2, ew3, eb3)


def _dec_body(lat_ref, w1_ref, b1_ref, w2_ref, b2_ref, w3_ref, b3_ref,
              out_ref, d2_ref):
    j = pl.program_id(0)

    @pl.when(j == 0)
    def _():
        d1 = jax.lax.dot(lat_ref[...].astype(BF16), w1_ref[...],
                         preferred_element_type=F32) + b1_ref[...]
        d1 = jnp.maximum(d1, 0.0).astype(BF16)
        d2 = jax.lax.dot(d1, w2_ref[...], preferred_element_type=F32) + b2_ref[...]
        d2_ref[...] = jnp.maximum(d2, 0.0).astype(BF16)

    out_ref[...] = (jax.lax.dot(d2_ref[...], w3_ref[...],
                                preferred_element_type=F32) + b3_ref[...])


def _decoder(lat, w1, b1, w2, b2, w3, b3):
    ntile = DEC_OUT // 2048
    return pl.pallas_call(
        _dec_body,
        grid=(ntile,),
        in_specs=[
            pl.BlockSpec((B, LAT), lambda j: (0, 0)),
            pl.BlockSpec((LAT, 1024), lambda j: (0, 0)),
            pl.BlockSpec((1, 1024), lambda j: (0, 0)),
            pl.BlockSpec((1024, 1024), lambda j: (0, 0)),
            pl.BlockSpec((1, 1024), lambda j: (0, 0)),
            pl.BlockSpec((1024, 2048), lambda j: (0, j)),
            pl.BlockSpec((1, 2048), lambda j: (0, j)),
        ],
        out_specs=pl.BlockSpec((B, 2048), lambda j: (0, j)),
        out_shape=jax.ShapeDtypeStruct((B, DEC_OUT), F32),
        scratch_shapes=[pltpu.VMEM((B, LAT), BF16)],
    )(lat, w1, b1, w2, b2, w3, b3)


def kernel(x, cats, se_W1, se_b1, se_W2, se_b2,
           enc_W1, enc_b1, enc_W2, enc_b2, enc_W3, enc_b3,
           dec_W1, dec_b1, dec_W2, dec_b2, dec_W3, dec_b3):
    cats32 = cats.astype(jnp.int32).reshape(128, 128)
    dest, te, nv = _routing(cats32)
    xt = _transpose(x)
    xt_bits = jax.lax.bitcast_convert_type(
        xt.reshape(NTOT, COLS // 2, 2), jnp.int32)
    buf_bits = _sc_scatter(xt_bits, dest.reshape(1, NTOT))
    buf = jax.lax.bitcast_convert_type(buf_bits, BF16).reshape(BUFROWS, COLS)

    ew1g = jnp.concatenate([enc_W1[:GEO], jnp.zeros((1, 256), F32)]).astype(BF16)
    ew1c = enc_W1[GEO:].astype(BF16)
    latent = _main(
        te.reshape(NT), nv.reshape(NT), buf,
        se_W1.astype(BF16), se_b1.reshape(E, 1, 256),
        se_W2.astype(BF16), se_b2.reshape(E, 1, SHAPE),
        ew1g, ew1c, enc_b1.reshape(1, 256),
        enc_W2.astype(BF16), enc_b2.reshape(1, 512),
        enc_W3.astype(BF16), enc_b3.reshape(1, LAT),
    )
    d = _decoder(latent, dec_W1.astype(BF16), dec_b1.reshape(1, LAT),
                 dec_W2.astype(BF16), dec_b2.reshape(1, LAT),
                 dec_W3.astype(BF16), dec_b3.reshape(1, DEC_OUT))
    return d.reshape(B, NPTS, GEO), latent


# speedup vs baseline: 1.7145x; 1.7145x over previous
import jax
import jax.numpy as jnp
from jax.experimental import pallas as pl
from jax.experimental.pallas import tpu as pltpu
from jax.experimental.pallas import tpu_sc as plsc

B = 8
NPTS = 2048
NTOT = B * NPTS
E = 16
GEO = 7
SHAPE = 128
T = 256
NT = NTOT // T + E
BUFROWS = NT * T
WORDS = 128
LAT = 1024
DEC_OUT = 14336
F32 = jnp.float32
BF16 = jnp.bfloat16


def _routing_body(cats_ref, dest_ref, te_ref, nv_ref):
    c = cats_ref[...]
    r_iota = jax.lax.broadcasted_iota(jnp.int32, (128, 128), 0)
    c_iota = jax.lax.broadcasted_iota(jnp.int32, (128, 128), 1)
    u_mat = (r_iota <= c_iota).astype(BF16)
    l_mat = (c_iota < r_iota).astype(BF16)

    counts, starts, ends = [], [], []
    run = jnp.float32(0.0)
    for e in range(E):
        cnt = jnp.sum((c == e).astype(F32))
        counts.append(cnt)
        starts.append(run)
        run = run + jnp.ceil(cnt / T) * T
        ends.append(run)

    dest = jnp.zeros((128, 128), F32)
    for e in range(E):
        m_f = (c == e).astype(F32)
        lane_pre = jax.lax.dot(m_f.astype(BF16), u_mat,
                               preferred_element_type=F32)
        rowsum = lane_pre[:, 127:128]
        rowpre = jax.lax.dot(l_mat, rowsum.astype(BF16),
                             preferred_element_type=F32)
        cum = lane_pre + rowpre
        dest = dest + m_f * (starts[e] + cum - 1.0)
    dest_ref[...] = dest.astype(jnp.int32)

    tv = jax.lax.broadcasted_iota(jnp.int32, (1, NT), 1).astype(F32) * T
    te = jnp.zeros((1, NT), F32)
    nv = jnp.zeros((1, NT), F32)
    for e in range(E):
        te = te + (tv >= ends[e]).astype(F32)
        inb = ((tv >= starts[e]) & (tv < ends[e])).astype(F32)
        nv = nv + inb * jnp.clip(counts[e] - (tv - starts[e]), 0.0, float(T))
    te_ref[...] = jnp.minimum(te, float(E - 1)).astype(jnp.int32)
    nv_ref[...] = nv.astype(jnp.int32)


def _routing(cats32):
    return pl.pallas_call(
        _routing_body,
        out_shape=[
            jax.ShapeDtypeStruct((128, 128), jnp.int32),
            jax.ShapeDtypeStruct((1, NT), jnp.int32),
            jax.ShapeDtypeStruct((1, NT), jnp.int32),
        ],
    )(cats32)


def _bf16_hi_bits(u):
    return (u + 0x7FFF + ((u >> 16) & 1)) & jnp.int32(-65536)


def _transpose_body(x_ref, xt_ref):
    b = pl.program_id(0)
    xb = x_ref[0]
    tr = xb.T
    btag = jnp.full((512, 1), 1.0, F32) * (b.astype(F32) + 1.0)
    codes = tr[:, GEO:]
    hi_src = jnp.concatenate(
        [tr[:, :GEO], btag, jnp.zeros((512, 120), F32)], axis=1)
    ul = jax.lax.bitcast_convert_type(codes, jnp.int32)
    uh = jax.lax.bitcast_convert_type(hi_src, jnp.int32)
    lo16 = jax.lax.shift_right_logical(_bf16_hi_bits(ul), 16)
    xt_ref[...] = lo16 | _bf16_hi_bits(uh)


def _transpose(x):
    return pl.pallas_call(
        _transpose_body,
        grid=(B, NPTS // 512),
        in_specs=[pl.BlockSpec((1, 135, 512), lambda b, j: (b, 0, j))],
        out_specs=pl.BlockSpec((512, WORDS), lambda b, j: (b * (NPTS // 512) + j, 0)),
        out_shape=jax.ShapeDtypeStruct((NTOT, WORDS), jnp.int32),
    )(x)


def _sc_scatter(xt_bits, dest):
    mesh = plsc.VectorSubcoreMesh(core_axis_name="c", subcore_axis_name="s")

    @pl.kernel(out_type=jax.ShapeDtypeStruct((BUFROWS, WORDS), jnp.int32),
               mesh=mesh)
    def k(x_hbm, i_hbm, o_hbm):
        def body(x_vmem, i_vmem):
            pltpu.sync_copy(x_vmem, o_hbm.at[i_vmem.at[0]])

        pltpu.emit_pipeline(
            body,
            grid=(NTOT // 128,),
            in_specs=[
                pl.BlockSpec((128, WORDS), lambda i: (i, 0)),
                pl.BlockSpec((1, 128), lambda i: (i, 0)),
            ],
            out_specs=[],
            core_axis_name=("c", "s"),
            dimension_semantics=(pltpu.PARALLEL,),
        )(x_hbm, i_hbm)

    return k(xt_bits, dest)


def _main_body(te_ref, nv_ref, buf_ref, w1_ref, b1_ref, w2_ref, b2_ref,
               ew1g_ref, ew1c_ref, eb1_ref, ew2_ref, eb2_ref, ew3_ref,
               eb3_ref, lat_ref):
    t = pl.program_id(0)

    @pl.when(t == 0)
    def _():
        lat_ref[...] = jnp.full((B, LAT), -jnp.inf, F32)

    w = buf_ref[...]
    codes = jax.lax.bitcast_convert_type(
        jax.lax.shift_left(w, 16), F32).astype(BF16)
    geob = jax.lax.bitcast_convert_type(w & jnp.int32(-65536), F32)
    h = jax.lax.dot(codes, w1_ref[0], preferred_element_type=F32) + b1_ref[0]
    h = jnp.maximum(h, 0.0).astype(BF16)
    sel = jax.lax.dot(h, w2_ref[0], preferred_element_type=F32) + b2_ref[0]
    geo8 = geob[:, 0:8].astype(BF16)
    t1 = (jax.lax.dot(geo8, ew1g_ref[...], preferred_element_type=F32)
          + jax.lax.dot(sel.astype(BF16), ew1c_ref[...], preferred_element_type=F32)
          + eb1_ref[...])
    t1 = jnp.maximum(t1, 0.0).astype(BF16)
    t2 = jax.lax.dot(t1, ew2_ref[...], preferred_element_type=F32) + eb2_ref[...]
    t2 = jnp.maximum(t2, 0.0).astype(BF16)
    t3 = jax.lax.dot(t2, ew3_ref[...], preferred_element_type=F32)

    nv = nv_ref[0, t]
    row = jax.lax.broadcasted_iota(jnp.int32, (T, 1), 0)
    validrow = row < nv
    btag = geob[:, 7:8]
    neg = jnp.float32(-jnp.inf)
    cands = []
    for b in range(B):
        mask = validrow & (btag == jnp.float32(b + 1))
        cands.append(jnp.max(jnp.where(mask, t3, neg), axis=0, keepdims=True))
    lat_ref[...] = jnp.maximum(lat_ref[...], jnp.concatenate(cands, axis=0))

    @pl.when(t == NT - 1)
    def _():
        lat_ref[...] = lat_ref[...] + eb3_ref[...]


def _main(te, nv, buf, w1, b1, w2, b2, ew1g, ew1c, eb1, ew2, eb2, ew3, eb3):
    grid_spec = pltpu.PrefetchScalarGridSpec(
        num_scalar_prefetch=2,
        grid=(NT,),
        in_specs=[
            pl.BlockSpec((T, WORDS), lambda t, te, nv: (t, 0)),
            pl.BlockSpec((1, SHAPE, 256), lambda t, te, nv: (te[0, t], 0, 0)),
            pl.BlockSpec((1, 1, 256), lambda t, te, nv: (te[0, t], 0, 0)),
            pl.BlockSpec((1, 256, SHAPE), lambda t, te, nv: (te[0, t], 0, 0)),
            pl.BlockSpec((1, 1, SHAPE), lambda t, te, nv: (te[0, t], 0, 0)),
            pl.BlockSpec((8, 256), lambda t, te, nv: (0, 0)),
            pl.BlockSpec((SHAPE, 256), lambda t, te, nv: (0, 0)),
            pl.BlockSpec((1, 256), lambda t, te, nv: (0, 0)),
            pl.BlockSpec((256, 512), lambda t, te, nv: (0, 0)),
            pl.BlockSpec((1, 512), lambda t, te, nv: (0, 0)),
            pl.BlockSpec((512, LAT), lambda t, te, nv: (0, 0)),
            pl.BlockSpec((1, LAT), lambda t, te, nv: (0, 0)),
        ],
        out_specs=pl.BlockSpec((B, LAT), lambda t, te, nv: (0, 0)),
    )
    return pl.pallas_call(
        _main_body,
        grid_spec=grid_spec,
        out_shape=jax.ShapeDtypeStruct((B, LAT), F32),
    )(te, nv, buf, w1, b1, w2, b2, ew1g, ew1c, eb1, ew2, eb2, ew3, eb3)


def _dec_body(lat_ref, w1_ref, b1_ref, w2_ref, b2_ref, w3_ref, b3_ref,
              out_ref, d2_ref):
    j = pl.program_id(0)

    @pl.when(j == 0)
    def _():
        d1 = jax.lax.dot(lat_ref[...].astype(BF16), w1_ref[...],
                         preferred_element_type=F32) + b1_ref[...]
        d1 = jnp.maximum(d1, 0.0).astype(BF16)
        d2 = jax.lax.dot(d1, w2_ref[...], preferred_element_type=F32) + b2_ref[...]
        d2_ref[...] = jnp.maximum(d2, 0.0).astype(BF16)

    out_ref[...] = (jax.lax.dot(d2_ref[...], w3_ref[...],
                                preferred_element_type=F32) + b3_ref[...])


def _decoder(lat, w1, b1, w2, b2, w3, b3):
    ntile = DEC_OUT // 2048
    return pl.pallas_call(
        _dec_body,
        grid=(ntile,),
        in_specs=[
            pl.BlockSpec((B, LAT), lambda j: (0, 0)),
            pl.BlockSpec((LAT, 1024), lambda j: (0, 0)),
            pl.BlockSpec((1, 1024), lambda j: (0, 0)),
            pl.BlockSpec((1024, 1024), lambda j: (0, 0)),
            pl.BlockSpec((1, 1024), lambda j: (0, 0)),
            pl.BlockSpec((1024, 2048), lambda j: (0, j)),
            pl.BlockSpec((1, 2048), lambda j: (0, j)),
        ],
        out_specs=pl.BlockSpec((B, 2048), lambda j: (0, j)),
        out_shape=jax.ShapeDtypeStruct((B, DEC_OUT), F32),
        scratch_shapes=[pltpu.VMEM((B, LAT), BF16)],
    )(lat, w1, b1, w2, b2, w3, b3)


def kernel(x, cats, se_W1, se_b1, se_W2, se_b2,
           enc_W1, enc_b1, enc_W2, enc_b2, enc_W3, enc_b3,
           dec_W1, dec_b1, dec_W2, dec_b2, dec_W3, dec_b3):
    cats32 = cats.astype(jnp.int32).reshape(128, 128)
    dest, te, nv = _routing(cats32)
    xt_bits = _transpose(x)
    buf = _sc_scatter(xt_bits, dest)

    ew1g = jnp.concatenate([enc_W1[:GEO], jnp.zeros((1, 256), F32)]).astype(BF16)
    ew1c = enc_W1[GEO:].astype(BF16)
    latent = _main(
        te, nv, buf,
        se_W1.astype(BF16), se_b1.reshape(E, 1, 256),
        se_W2.astype(BF16), se_b2.reshape(E, 1, SHAPE),
        ew1g, ew1c, enc_b1.reshape(1, 256),
        enc_W2.astype(BF16), enc_b2.reshape(1, 512),
        enc_W3.astype(BF16), enc_b3.reshape(1, LAT),
    )
    d = _decoder(latent, dec_W1.astype(BF16), dec_b1.reshape(1, LAT),
                 dec_W2.astype(BF16), dec_b2.reshape(1, LAT),
                 dec_W3.astype(BF16), dec_b3.reshape(1, DEC_OUT))
    return d.reshape(B, NPTS, GEO), latent


# speedup vs baseline: 1.8649x; 1.0877x over previous
import jax
import jax.numpy as jnp
from jax.experimental import pallas as pl
from jax.experimental.pallas import tpu as pltpu
from jax.experimental.pallas import tpu_sc as plsc

B = 8
NPTS = 2048
NTOT = B * NPTS
E = 16
GEO = 7
SHAPE = 128
T = 256
NT = NTOT // T + E
BUFROWS = NT * T
WORDS = 128
LAT = 1024
DEC_OUT = 14336
DTILE = 2048
NDEC = DEC_OUT // DTILE
F32 = jnp.float32
BF16 = jnp.bfloat16


def _bf16_hi_bits(u):
    return (u + 0x7FFF + ((u >> 16) & 1)) & jnp.int32(-65536)


def _prep_body(cats_ref, x_ref, dest_ref, te_ref, nv_ref, xt_ref):
    t = pl.program_id(0)

    @pl.when(t == 0)
    def _routing():
        c = cats_ref[...]
        r_iota = jax.lax.broadcasted_iota(jnp.int32, (128, 128), 0)
        c_iota = jax.lax.broadcasted_iota(jnp.int32, (128, 128), 1)
        u_mat = (r_iota <= c_iota).astype(BF16)
        l_mat = (c_iota < r_iota).astype(BF16)

        counts, starts, ends = [], [], []
        run = jnp.float32(0.0)
        for e in range(E):
            cnt = jnp.sum((c == e).astype(F32))
            counts.append(cnt)
            starts.append(run)
            run = run + jnp.ceil(cnt / T) * T
            ends.append(run)

        dest = jnp.zeros((128, 128), F32)
        for e in range(E):
            m_f = (c == e).astype(F32)
            lane_pre = jax.lax.dot(m_f.astype(BF16), u_mat,
                                   preferred_element_type=F32)
            rowsum = lane_pre[:, 127:128]
            rowpre = jax.lax.dot(l_mat, rowsum.astype(BF16),
                                 preferred_element_type=F32)
            cum = lane_pre + rowpre
            dest = dest + m_f * (starts[e] + cum - 1.0)
        dest_ref[...] = dest.astype(jnp.int32)

        tv = jax.lax.broadcasted_iota(jnp.int32, (1, NT), 1).astype(F32) * T
        te = jnp.zeros((1, NT), F32)
        nv = jnp.zeros((1, NT), F32)
        for e in range(E):
            te = te + (tv >= ends[e]).astype(F32)
            inb = ((tv >= starts[e]) & (tv < ends[e])).astype(F32)
            nv = nv + inb * jnp.clip(counts[e] - (tv - starts[e]), 0.0, float(T))
        te_ref[...] = jnp.minimum(te, float(E - 1)).astype(jnp.int32)
        nv_ref[...] = nv.astype(jnp.int32)

    @pl.when(t > 0)
    def _transpose():
        b = (t - 1) // (NPTS // 512)
        xb = x_ref[0]
        tr = xb.T
        btag = jnp.full((512, 1), 1.0, F32) * (b.astype(F32) + 1.0)
        codes = tr[:, GEO:]
        hi_src = jnp.concatenate(
            [tr[:, :GEO], btag, jnp.zeros((512, 120), F32)], axis=1)
        ul = jax.lax.bitcast_convert_type(codes, jnp.int32)
        uh = jax.lax.bitcast_convert_type(hi_src, jnp.int32)
        lo16 = jax.lax.shift_right_logical(_bf16_hi_bits(ul), 16)
        xt_ref[...] = lo16 | _bf16_hi_bits(uh)


def _prep(cats32, x):
    nj = NPTS // 512
    return pl.pallas_call(
        _prep_body,
        grid=(1 + B * nj,),
        in_specs=[
            pl.BlockSpec((128, 128), lambda t: (0, 0)),
            pl.BlockSpec((1, 135, 512),
                         lambda t: ((jnp.maximum(t, 1) - 1) // nj, 0,
                                    (jnp.maximum(t, 1) - 1) % nj)),
        ],
        out_specs=[
            pl.BlockSpec((128, 128), lambda t: (0, 0)),
            pl.BlockSpec((1, NT), lambda t: (0, 0)),
            pl.BlockSpec((1, NT), lambda t: (0, 0)),
            pl.BlockSpec((512, WORDS), lambda t: (jnp.maximum(t, 1) - 1, 0)),
        ],
        out_shape=[
            jax.ShapeDtypeStruct((128, 128), jnp.int32),
            jax.ShapeDtypeStruct((1, NT), jnp.int32),
            jax.ShapeDtypeStruct((1, NT), jnp.int32),
            jax.ShapeDtypeStruct((NTOT, WORDS), jnp.int32),
        ],
    )(cats32, x)


def _sc_scatter(xt_bits, dest):
    mesh = plsc.VectorSubcoreMesh(core_axis_name="c", subcore_axis_name="s")

    @pl.kernel(out_type=jax.ShapeDtypeStruct((BUFROWS, WORDS), jnp.int32),
               mesh=mesh)
    def k(x_hbm, i_hbm, o_hbm):
        def body(x_vmem, i_vmem):
            pltpu.sync_copy(x_vmem, o_hbm.at[i_vmem.at[0]])

        pltpu.emit_pipeline(
            body,
            grid=(NTOT // 128,),
            in_specs=[
                pl.BlockSpec((128, WORDS), lambda i: (i, 0)),
                pl.BlockSpec((1, 128), lambda i: (i, 0)),
            ],
            out_specs=[],
            core_axis_name=("c", "s"),
            dimension_semantics=(pltpu.PARALLEL,),
        )(x_hbm, i_hbm)

    return k(xt_bits, dest)


def _main_body(te_ref, nv_ref, buf_ref, w1_ref, b1_ref, w2_ref, b2_ref,
               ew1g_ref, ew1c_ref, eb1_ref, ew2_ref, eb2_ref, ew3_ref,
               eb3_ref, dw1_ref, db1_ref, dw2_ref, db2_ref, dw3_ref, db3_ref,
               lat_ref, out_ref, d2_ref):
    t = pl.program_id(0)

    @pl.when(t == 0)
    def _():
        lat_ref[...] = jnp.full((B, LAT), -jnp.inf, F32)

    nv = nv_ref[0, jnp.minimum(t, NT - 1)]

    @pl.when((t < NT) & (nv > 0))
    def _tile():
        w = buf_ref[...]
        codes = jax.lax.bitcast_convert_type(
            jax.lax.shift_left(w, 16), F32).astype(BF16)
        geob = jax.lax.bitcast_convert_type(w & jnp.int32(-65536), F32)
        h = jax.lax.dot(codes, w1_ref[0], preferred_element_type=F32) + b1_ref[0]
        h = jnp.maximum(h, 0.0).astype(BF16)
        sel = jax.lax.dot(h, w2_ref[0], preferred_element_type=F32) + b2_ref[0]
        geo8 = geob[:, 0:8].astype(BF16)
        t1 = (jax.lax.dot(geo8, ew1g_ref[...], preferred_element_type=F32)
              + jax.lax.dot(sel.astype(BF16), ew1c_ref[...],
                            preferred_element_type=F32)
              + eb1_ref[...])
        t1 = jnp.maximum(t1, 0.0).astype(BF16)
        t2 = jax.lax.dot(t1, ew2_ref[...], preferred_element_type=F32) + eb2_ref[...]
        t2 = jnp.maximum(t2, 0.0).astype(BF16)
        t3 = jax.lax.dot(t2, ew3_ref[...], preferred_element_type=F32)

        row = jax.lax.broadcasted_iota(jnp.int32, (T, 1), 0)
        validrow = row < nv
        btag = geob[:, 7:8]
        bmin = jnp.min(jnp.where(validrow, btag, 9.0))
        bmax = jnp.max(jnp.where(validrow, btag, 0.0))
        neg = jnp.float32(-jnp.inf)
        for b in range(B):
            bf = jnp.float32(b + 1)

            @pl.when((bmin <= bf) & (bf <= bmax))
            def _upd():
                mask = validrow & (btag == bf)
                cand = jnp.max(jnp.where(mask, t3, neg), axis=0, keepdims=True)
                lat_ref[b:b + 1, :] = jnp.maximum(lat_ref[b:b + 1, :], cand)

    @pl.when(t == NT - 1)
    def _():
        lat_ref[...] = lat_ref[...] + eb3_ref[...]

    @pl.when(t == NT)
    def _():
        d1 = jax.lax.dot(lat_ref[...].astype(BF16), dw1_ref[...],
                         preferred_element_type=F32) + db1_ref[...]
        d1 = jnp.maximum(d1, 0.0).astype(BF16)
        d2 = jax.lax.dot(d1, dw2_ref[...], preferred_element_type=F32) + db2_ref[...]
        d2_ref[...] = jnp.maximum(d2, 0.0).astype(BF16)

    @pl.when(t >= NT)
    def _():
        out_ref[...] = (jax.lax.dot(d2_ref[...], dw3_ref[...],
                                    preferred_element_type=F32) + db3_ref[...])


def _main(te, nv, buf, w1, b1, w2, b2, ew1g, ew1c, eb1, ew2, eb2, ew3, eb3,
          dw1, db1, dw2, db2, dw3, db3):
    def tile_idx(t, te, nv):
        return (jnp.minimum(t, NT - 1), 0)

    def exp_idx3(t, te, nv):
        return (te[0, jnp.minimum(t, NT - 1)], 0, 0)

    def dec_idx(t, te, nv):
        return (0, jnp.maximum(t - NT, 0))

    const2 = lambda t, te, nv: (0, 0)
    grid_spec = pltpu.PrefetchScalarGridSpec(
        num_scalar_prefetch=2,
        grid=(NT + NDEC,),
        in_specs=[
            pl.BlockSpec((T, WORDS), tile_idx),
            pl.BlockSpec((1, SHAPE, 256), exp_idx3),
            pl.BlockSpec((1, 1, 256), exp_idx3),
            pl.BlockSpec((1, 256, SHAPE), exp_idx3),
            pl.BlockSpec((1, 1, SHAPE), exp_idx3),
            pl.BlockSpec((8, 256), const2),
            pl.BlockSpec((SHAPE, 256), const2),
            pl.BlockSpec((1, 256), const2),
            pl.BlockSpec((256, 512), const2),
            pl.BlockSpec((1, 512), const2),
            pl.BlockSpec((512, LAT), const2),
            pl.BlockSpec((1, LAT), const2),
            pl.BlockSpec((LAT, LAT), const2),
            pl.BlockSpec((1, LAT), const2),
            pl.BlockSpec((LAT, LAT), const2),
            pl.BlockSpec((1, LAT), const2),
            pl.BlockSpec((LAT, DTILE), dec_idx),
            pl.BlockSpec((1, DTILE), dec_idx),
        ],
        out_specs=[
            pl.BlockSpec((B, LAT), const2),
            pl.BlockSpec((B, DTILE), dec_idx),
        ],
        scratch_shapes=[pltpu.VMEM((B, LAT), BF16)],
    )
    return pl.pallas_call(
        _main_body,
        grid_spec=grid_spec,
        out_shape=[
            jax.ShapeDtypeStruct((B, LAT), F32),
            jax.ShapeDtypeStruct((B, DEC_OUT), F32),
        ],
    )(te, nv, buf, w1, b1, w2, b2, ew1g, ew1c, eb1, ew2, eb2, ew3, eb3,
      dw1, db1, dw2, db2, dw3, db3)


def kernel(x, cats, se_W1, se_b1, se_W2, se_b2,
           enc_W1, enc_b1, enc_W2, enc_b2, enc_W3, enc_b3,
           dec_W1, dec_b1, dec_W2, dec_b2, dec_W3, dec_b3):
    cats32 = cats.astype(jnp.int32).reshape(128, 128)
    dest, te, nv, xt_bits = _prep(cats32, x)
    buf = _sc_scatter(xt_bits, dest)

    ew1g = jnp.concatenate([enc_W1[:GEO], jnp.zeros((1, 256), F32)]).astype(BF16)
    ew1c = enc_W1[GEO:].astype(BF16)
    latent, d = _main(
        te, nv, buf,
        se_W1.astype(BF16), se_b1.reshape(E, 1, 256),
        se_W2.astype(BF16), se_b2.reshape(E, 1, SHAPE),
        ew1g, ew1c, enc_b1.reshape(1, 256),
        enc_W2.astype(BF16), enc_b2.reshape(1, 512),
        enc_W3.astype(BF16), enc_b3.reshape(1, LAT),
        dec_W1.astype(BF16), dec_b1.reshape(1, LAT),
        dec_W2.astype(BF16), dec_b2.reshape(1, LAT),
        dec_W3.astype(BF16), dec_b3.reshape(1, DEC_OUT),
    )
    return d.reshape(B, NPTS, GEO), latent


# speedup vs baseline: 2.1484x; 1.1520x over previous
import jax
import jax.numpy as jnp
from jax.experimental import pallas as pl
from jax.experimental.pallas import tpu as pltpu
from jax.experimental.pallas import tpu_sc as plsc

B = 8
NPTS = 2048
NTOT = B * NPTS
E = 16
GEO = 7
SHAPE = 128
T = 256
NT = NTOT // T + E
BUFROWS = NT * T
WORDS = 128
LAT = 1024
DEC_OUT = 14336
DTILE = 2048
NDEC = DEC_OUT // DTILE
F32 = jnp.float32
BF16 = jnp.bfloat16


def _bf16_hi_bits(u):
    return (u + 0x7FFF + ((u >> 16) & 1)) & jnp.int32(-65536)


def _prep_body(cats_ref, x_ref, dest_ref, te_ref, nv_ref, xt_ref):
    t = pl.program_id(0)

    @pl.when(t == 0)
    def _routing():
        c = cats_ref[...]
        r_iota = jax.lax.broadcasted_iota(jnp.int32, (128, 128), 0)
        c_iota = jax.lax.broadcasted_iota(jnp.int32, (128, 128), 1)
        u_mat = (r_iota <= c_iota).astype(BF16)
        l_mat = (c_iota < r_iota).astype(BF16)

        counts, starts, ends = [], [], []
        run = jnp.float32(0.0)
        for e in range(E):
            cnt = jnp.sum((c == e).astype(F32))
            counts.append(cnt)
            starts.append(run)
            run = run + jnp.ceil(cnt / T) * T
            ends.append(run)

        dest = jnp.zeros((128, 128), F32)
        for e in range(E):
            m_f = (c == e).astype(F32)
            lane_pre = jax.lax.dot(m_f.astype(BF16), u_mat,
                                   preferred_element_type=F32)
            rowsum = lane_pre[:, 127:128]
            rowpre = jax.lax.dot(l_mat, rowsum.astype(BF16),
                                 preferred_element_type=F32)
            cum = lane_pre + rowpre
            dest = dest + m_f * (starts[e] + cum - 1.0)
        dest_ref[...] = dest.astype(jnp.int32)

        tv = jax.lax.broadcasted_iota(jnp.int32, (1, NT), 1).astype(F32) * T
        te = jnp.zeros((1, NT), F32)
        nv = jnp.zeros((1, NT), F32)
        for e in range(E):
            te = te + (tv >= ends[e]).astype(F32)
            inb = ((tv >= starts[e]) & (tv < ends[e])).astype(F32)
            nv = nv + inb * jnp.clip(counts[e] - (tv - starts[e]), 0.0, float(T))
        te_ref[...] = jnp.minimum(te, float(E - 1)).astype(jnp.int32)
        nv_ref[...] = nv.astype(jnp.int32)

    @pl.when(t > 0)
    def _transpose():
        b = t - 1
        xb = x_ref[0]
        tr = xb.T
        btag = jnp.full((NPTS, 1), 1.0, F32) * (b.astype(F32) + 1.0)
        codes = tr[:, GEO:]
        hi_src = jnp.concatenate(
            [tr[:, :GEO], btag, jnp.zeros((NPTS, 120), F32)], axis=1)
        ul = jax.lax.bitcast_convert_type(codes, jnp.int32)
        uh = jax.lax.bitcast_convert_type(hi_src, jnp.int32)
        lo16 = jax.lax.shift_right_logical(_bf16_hi_bits(ul), 16)
        xt_ref[...] = lo16 | _bf16_hi_bits(uh)


def _prep(cats32, x):
    return pl.pallas_call(
        _prep_body,
        grid=(1 + B,),
        in_specs=[
            pl.BlockSpec((128, 128), lambda t: (0, 0)),
            pl.BlockSpec((1, 135, NPTS),
                         lambda t: (jnp.maximum(t, 1) - 1, 0, 0)),
        ],
        out_specs=[
            pl.BlockSpec((128, 128), lambda t: (0, 0)),
            pl.BlockSpec((1, NT), lambda t: (0, 0)),
            pl.BlockSpec((1, NT), lambda t: (0, 0)),
            pl.BlockSpec((NPTS, WORDS), lambda t: (jnp.maximum(t, 1) - 1, 0)),
        ],
        out_shape=[
            jax.ShapeDtypeStruct((128, 128), jnp.int32),
            jax.ShapeDtypeStruct((1, NT), jnp.int32),
            jax.ShapeDtypeStruct((1, NT), jnp.int32),
            jax.ShapeDtypeStruct((NTOT, WORDS), jnp.int32),
        ],
    )(cats32, x)


def _sc_scatter(xt_bits, dest):
    mesh = plsc.VectorSubcoreMesh(core_axis_name="c", subcore_axis_name="s")

    @pl.kernel(out_type=jax.ShapeDtypeStruct((BUFROWS, WORDS), jnp.int32),
               mesh=mesh)
    def k(x_hbm, i_hbm, o_hbm):
        def body(x_vmem, i_vmem):
            pltpu.sync_copy(x_vmem, o_hbm.at[i_vmem.at[0]])

        pltpu.emit_pipeline(
            body,
            grid=(NTOT // 128,),
            in_specs=[
                pl.BlockSpec((128, WORDS), lambda i: (i, 0)),
                pl.BlockSpec((1, 128), lambda i: (i, 0)),
            ],
            out_specs=[],
            core_axis_name=("c", "s"),
            dimension_semantics=(pltpu.PARALLEL,),
        )(x_hbm, i_hbm)

    return k(xt_bits, dest)


def _main_body(te_ref, nv_ref, buf_ref, w1_ref, b1_ref, w2_ref, b2_ref,
               ew1g_ref, ew1c_ref, eb1_ref, ew2_ref, eb2_ref, ew3_ref,
               eb3_ref, dw1_ref, db1_ref, dw2_ref, db2_ref, dw3_ref, db3_ref,
               lat_ref, out_ref, d2_ref):
    t = pl.program_id(0)

    @pl.when(t == 0)
    def _():
        lat_ref[...] = jnp.full((B, LAT), -jnp.inf, F32)

    nv = nv_ref[0, jnp.minimum(t, NT - 1)]

    @pl.when((t < NT) & (nv > 0))
    def _tile():
        w = buf_ref[...]
        codes = jax.lax.bitcast_convert_type(
            jax.lax.shift_left(w, 16), F32).astype(BF16)
        geob = jax.lax.bitcast_convert_type(w & jnp.int32(-65536), F32)
        h = jax.lax.dot(codes, w1_ref[0], preferred_element_type=F32) + b1_ref[0]
        h = jnp.maximum(h, 0.0).astype(BF16)
        sel = jax.lax.dot(h, w2_ref[0], preferred_element_type=F32) + b2_ref[0]
        geo8 = geob[:, 0:8].astype(BF16)
        t1 = (jax.lax.dot(geo8, ew1g_ref[...], preferred_element_type=F32)
              + jax.lax.dot(sel.astype(BF16), ew1c_ref[...],
                            preferred_element_type=F32)
              + eb1_ref[...])
        t1 = jnp.maximum(t1, 0.0).astype(BF16)
        t2 = jax.lax.dot(t1, ew2_ref[...], preferred_element_type=F32) + eb2_ref[...]
        t2 = jnp.maximum(t2, 0.0).astype(BF16)
        t3 = jax.lax.dot(t2, ew3_ref[...], preferred_element_type=F32)

        row = jax.lax.broadcasted_iota(jnp.int32, (T, 1), 0)
        validrow = row < nv
        btag = geob[:, 7:8]
        bmin = jnp.min(jnp.where(validrow, btag, 9.0))
        bmax = jnp.max(jnp.where(validrow, btag, 0.0))
        neg = jnp.float32(-jnp.inf)
        for b in range(B):
            bf = jnp.float32(b + 1)

            @pl.when((bmin <= bf) & (bf <= bmax))
            def _upd():
                mask = validrow & (btag == bf)
                cand = jnp.max(jnp.where(mask, t3, neg), axis=0, keepdims=True)
                lat_ref[b:b + 1, :] = jnp.maximum(lat_ref[b:b + 1, :], cand)

    @pl.when(t == NT - 1)
    def _():
        lat_ref[...] = lat_ref[...] + eb3_ref[...]

    @pl.when(t == NT)
    def _():
        d1 = jax.lax.dot(lat_ref[...].astype(BF16), dw1_ref[...],
                         preferred_element_type=F32) + db1_ref[...]
        d1 = jnp.maximum(d1, 0.0).astype(BF16)
        d2 = jax.lax.dot(d1, dw2_ref[...], preferred_element_type=F32) + db2_ref[...]
        d2_ref[...] = jnp.maximum(d2, 0.0).astype(BF16)

    @pl.when(t >= NT)
    def _():
        out_ref[...] = (jax.lax.dot(d2_ref[...], dw3_ref[...].astype(BF16),
                                    preferred_element_type=F32) + db3_ref[...])


def _main(te, nv, buf, w1, b1, w2, b2, ew1g, ew1c, eb1, ew2, eb2, ew3, eb3,
          dw1, db1, dw2, db2, dw3, db3):
    def tile_idx(t, te, nv):
        return (jnp.minimum(t, NT - 1), 0)

    def exp_idx3(t, te, nv):
        return (te[0, jnp.minimum(t, NT - 1)], 0, 0)

    def dec_idx(t, te, nv):
        return (0, jnp.maximum(t - NT, 0))

    const2 = lambda t, te, nv: (0, 0)
    grid_spec = pltpu.PrefetchScalarGridSpec(
        num_scalar_prefetch=2,
        grid=(NT + NDEC,),
        in_specs=[
            pl.BlockSpec((T, WORDS), tile_idx),
            pl.BlockSpec((1, SHAPE, 256), exp_idx3),
            pl.BlockSpec((1, 1, 256), exp_idx3),
            pl.BlockSpec((1, 256, SHAPE), exp_idx3),
            pl.BlockSpec((1, 1, SHAPE), exp_idx3),
            pl.BlockSpec((8, 256), const2),
            pl.BlockSpec((SHAPE, 256), const2),
            pl.BlockSpec((1, 256), const2),
            pl.BlockSpec((256, 512), const2),
            pl.BlockSpec((1, 512), const2),
            pl.BlockSpec((512, LAT), const2),
            pl.BlockSpec((1, LAT), const2),
            pl.BlockSpec((LAT, LAT), const2),
            pl.BlockSpec((1, LAT), const2),
            pl.BlockSpec((LAT, LAT), const2),
            pl.BlockSpec((1, LAT), const2),
            pl.BlockSpec((LAT, DTILE), dec_idx),
            pl.BlockSpec((1, DTILE), dec_idx),
        ],
        out_specs=[
            pl.BlockSpec((B, LAT), const2),
            pl.BlockSpec((B, DTILE), dec_idx),
        ],
        scratch_shapes=[pltpu.VMEM((B, LAT), BF16)],
    )
    return pl.pallas_call(
        _main_body,
        grid_spec=grid_spec,
        out_shape=[
            jax.ShapeDtypeStruct((B, LAT), F32),
            jax.ShapeDtypeStruct((B, DEC_OUT), F32),
        ],
    )(te, nv, buf, w1, b1, w2, b2, ew1g, ew1c, eb1, ew2, eb2, ew3, eb3,
      dw1, db1, dw2, db2, dw3, db3)


def kernel(x, cats, se_W1, se_b1, se_W2, se_b2,
           enc_W1, enc_b1, enc_W2, enc_b2, enc_W3, enc_b3,
           dec_W1, dec_b1, dec_W2, dec_b2, dec_W3, dec_b3):
    cats32 = cats.astype(jnp.int32).reshape(128, 128)
    dest, te, nv, xt_bits = _prep(cats32, x)
    buf = _sc_scatter(xt_bits, dest)

    ew1g = jnp.concatenate([enc_W1[:GEO], jnp.zeros((1, 256), F32)]).astype(BF16)
    ew1c = enc_W1[GEO:].astype(BF16)
    latent, d = _main(
        te, nv, buf,
        se_W1.astype(BF16), se_b1.reshape(E, 1, 256),
        se_W2.astype(BF16), se_b2.reshape(E, 1, SHAPE),
        ew1g, ew1c, enc_b1.reshape(1, 256),
        enc_W2.astype(BF16), enc_b2.reshape(1, 512),
        enc_W3.astype(BF16), enc_b3.reshape(1, LAT),
        dec_W1.astype(BF16), dec_b1.reshape(1, LAT),
        dec_W2.astype(BF16), dec_b2.reshape(1, LAT),
        dec_W3, dec_b3.reshape(1, DEC_OUT),
    )
    return d.reshape(B, NPTS, GEO), latent


# speedup vs baseline: 2.4518x; 1.1413x over previous
import jax
import jax.numpy as jnp
from jax.experimental import pallas as pl
from jax.experimental.pallas import tpu as pltpu
from jax.experimental.pallas import tpu_sc as plsc

B = 8
NPTS = 2048
NTOT = B * NPTS
E = 16
GEO = 7
SHAPE = 128
T = 512
NT = NTOT // T + E
BUFROWS = NT * T
WORDS = 128
PCHUNK = 256
LAT = 1024
DEC_OUT = 14336
DTILE = 2048
NDEC = DEC_OUT // DTILE
F32 = jnp.float32
BF16 = jnp.bfloat16


def _bf16_hi_bits(u):
    return (u + 0x7FFF + ((u >> 16) & 1)) & jnp.int32(-65536)


def _prep_body(cats_ref, x_ref, sw2_ref, sb2_ref, ew1_ref, dest_ref, te_ref, nv_ref, xt_ref, fw_ref, fb_ref, ew1g_ref):
    t = pl.program_id(0)

    @pl.when(t == 0)
    def _routing():
        c = cats_ref[...]
        r_iota = jax.lax.broadcasted_iota(jnp.int32, (128, 128), 0)
        c_iota = jax.lax.broadcasted_iota(jnp.int32, (128, 128), 1)
        u_mat = (r_iota <= c_iota).astype(BF16)
        l_mat = (c_iota < r_iota).astype(BF16)

        counts, starts, ends = [], [], []
        run = jnp.float32(0.0)
        for e in range(E):
            cnt = jnp.sum((c == e).astype(F32))
            counts.append(cnt)
            starts.append(run)
            run = run + jnp.ceil(cnt / T) * T
            ends.append(run)

        dest = jnp.zeros((128, 128), F32)
        for e in range(E):
            m_f = (c == e).astype(F32)
            lane_pre = jax.lax.dot(m_f.astype(BF16), u_mat,
                                   preferred_element_type=F32)
            rowsum = lane_pre[:, 127:128]
            rowpre = jax.lax.dot(l_mat, rowsum.astype(BF16),
                                 preferred_element_type=F32)
            cum = lane_pre + rowpre
            dest = dest + m_f * (starts[e] + cum - 1.0)
        dest_ref[...] = dest.astype(jnp.int32)

        tv = jax.lax.broadcasted_iota(jnp.int32, (1, NT), 1).astype(F32) * T
        te = jnp.zeros((1, NT), F32)
        nv = jnp.zeros((1, NT), F32)
        for e in range(E):
            te = te + (tv >= ends[e]).astype(F32)
            inb = ((tv >= starts[e]) & (tv < ends[e])).astype(F32)
            nv = nv + inb * jnp.clip(counts[e] - (tv - starts[e]), 0.0, float(T))
        te_ref[...] = jnp.minimum(te, float(E - 1)).astype(jnp.int32)
        nv_ref[...] = nv.astype(jnp.int32)

        w1c = ew1_ref[GEO:, :].astype(BF16)
        ew1g_ref[...] = jnp.concatenate(
            [ew1_ref[:GEO, :], jnp.zeros((1, 256), F32)], axis=0).astype(BF16)
        for e in range(E):
            fw_ref[e] = jax.lax.dot(sw2_ref[e].astype(BF16), w1c,
                                    preferred_element_type=F32).astype(BF16)
            fb_ref[e] = jax.lax.dot(sb2_ref[e].astype(BF16), w1c,
                                    preferred_element_type=F32)

    @pl.when(t > 0)
    def _transpose():
        x3 = x_ref[...]
        for b in range(B):
            tr = x3[:, b, :].T
            btag = jnp.full((PCHUNK, 1), float(b + 1), F32)
            codes = tr[:, GEO:]
            hi_src = jnp.concatenate(
                [tr[:, :GEO], btag, jnp.zeros((PCHUNK, 120), F32)], axis=1)
            ul = jax.lax.bitcast_convert_type(codes, jnp.int32)
            uh = jax.lax.bitcast_convert_type(hi_src, jnp.int32)
            lo16 = jax.lax.shift_right_logical(_bf16_hi_bits(ul), 16)
            xt_ref[b] = lo16 | _bf16_hi_bits(uh)


def _prep(cats32, xp, se_W2, se_b2, enc_W1):
    return pl.pallas_call(
        _prep_body,
        grid=(1 + NPTS // PCHUNK,),
        in_specs=[
            pl.BlockSpec((128, 128), lambda t: (0, 0)),
            pl.BlockSpec((135, B, PCHUNK),
                         lambda t: (0, 0, jnp.maximum(t, 1) - 1)),
            pl.BlockSpec((E, 256, SHAPE), lambda t: (0, 0, 0)),
            pl.BlockSpec((E, 1, SHAPE), lambda t: (0, 0, 0)),
            pl.BlockSpec((135, 256), lambda t: (0, 0)),
        ],
        out_specs=[
            pl.BlockSpec((128, 128), lambda t: (0, 0)),
            pl.BlockSpec((1, NT), lambda t: (0, 0)),
            pl.BlockSpec((1, NT), lambda t: (0, 0)),
            pl.BlockSpec((B, PCHUNK, WORDS),
                         lambda t: (0, jnp.maximum(t, 1) - 1, 0)),
            pl.BlockSpec((E, 256, 256), lambda t: (0, 0, 0)),
            pl.BlockSpec((E, 1, 256), lambda t: (0, 0, 0)),
            pl.BlockSpec((8, 256), lambda t: (0, 0)),
        ],
        out_shape=[
            jax.ShapeDtypeStruct((128, 128), jnp.int32),
            jax.ShapeDtypeStruct((1, NT), jnp.int32),
            jax.ShapeDtypeStruct((1, NT), jnp.int32),
            jax.ShapeDtypeStruct((B, NPTS, WORDS), jnp.int32),
            jax.ShapeDtypeStruct((E, 256, 256), BF16),
            jax.ShapeDtypeStruct((E, 1, 256), F32),
            jax.ShapeDtypeStruct((8, 256), BF16),
        ],
    )(cats32, xp, se_W2, se_b2, enc_W1)


def _sc_scatter(xt_bits, dest):
    mesh = plsc.VectorSubcoreMesh(core_axis_name="c", subcore_axis_name="s")

    @pl.kernel(out_type=jax.ShapeDtypeStruct((BUFROWS, WORDS), jnp.int32),
               mesh=mesh)
    def k(x_hbm, i_hbm, o_hbm):
        def body(x_vmem, i_vmem):
            pltpu.sync_copy(x_vmem, o_hbm.at[i_vmem.at[0]])

        pltpu.emit_pipeline(
            body,
            grid=(NTOT // 128,),
            in_specs=[
                pl.BlockSpec((128, WORDS), lambda i: (i, 0)),
                pl.BlockSpec((1, 128), lambda i: (i, 0)),
            ],
            out_specs=[],
            core_axis_name=("c", "s"),
            dimension_semantics=(pltpu.PARALLEL,),
        )(x_hbm, i_hbm)

    return k(xt_bits, dest)


def _main_body(te_ref, nv_ref, buf_ref, w1_ref, b1_ref, fw_ref, fb_ref,
               ew1g_ref, eb1_ref, ew2_ref, eb2_ref, ew3_ref,
               eb3_ref, dw1_ref, db1_ref, dw2_ref, db2_ref, dw3_ref, db3_ref,
               lat_ref, out_ref, d2_ref):
    t = pl.program_id(0)

    @pl.when(t == 0)
    def _():
        lat_ref[...] = jnp.full((B, LAT), -jnp.inf, F32)

    nv = nv_ref[0, jnp.minimum(t, NT - 1)]

    @pl.when((t < NT) & (nv > 0))
    def _tile():
        w = buf_ref[...]
        codes = jax.lax.bitcast_convert_type(
            jax.lax.shift_left(w, 16), F32).astype(BF16)
        geob = jax.lax.bitcast_convert_type(w & jnp.int32(-65536), F32)
        h = jax.lax.dot(codes, w1_ref[0], preferred_element_type=F32) + b1_ref[0]
        h = jnp.maximum(h, 0.0).astype(BF16)
        geo8 = geob[:, 0:8].astype(BF16)
        t1 = (jax.lax.dot(geo8, ew1g_ref[...], preferred_element_type=F32)
              + jax.lax.dot(h, fw_ref[0], preferred_element_type=F32)
              + eb1_ref[...] + fb_ref[0])
        t1 = jnp.maximum(t1, 0.0).astype(BF16)
        t2 = jax.lax.dot(t1, ew2_ref[...], preferred_element_type=F32) + eb2_ref[...]
        t2 = jnp.maximum(t2, 0.0).astype(BF16)
        t3 = jax.lax.dot(t2, ew3_ref[...],
                         preferred_element_type=F32).astype(BF16)

        row = jax.lax.broadcasted_iota(jnp.int32, (T, 1), 0)
        validrow = row < nv
        btag = geob[:, 7:8]
        bmin = jnp.min(jnp.where(validrow, btag, 9.0))
        bmax = jnp.max(jnp.where(validrow, btag, 0.0))
        neg = jnp.bfloat16(-jnp.inf)
        for b in range(B):
            bf = jnp.float32(b + 1)

            @pl.when((bmin <= bf) & (bf <= bmax))
            def _upd():
                mask = validrow & (btag == bf)
                cand = jnp.max(jnp.where(mask, t3, neg), axis=0, keepdims=True)
                lat_ref[b:b + 1, :] = jnp.maximum(lat_ref[b:b + 1, :],
                                                  cand.astype(F32))

    @pl.when(t == NT - 1)
    def _():
        lat_ref[...] = lat_ref[...] + eb3_ref[...]

    @pl.when(t == NT)
    def _():
        d1 = jax.lax.dot(lat_ref[...].astype(BF16), dw1_ref[...],
                         preferred_element_type=F32) + db1_ref[...]
        d1 = jnp.maximum(d1, 0.0).astype(BF16)
        d2 = jax.lax.dot(d1, dw2_ref[...], preferred_element_type=F32) + db2_ref[...]
        d2_ref[...] = jnp.maximum(d2, 0.0).astype(BF16)

    @pl.when(t >= NT)
    def _():
        out_ref[...] = (jax.lax.dot(d2_ref[...], dw3_ref[...].astype(BF16),
                                    preferred_element_type=F32) + db3_ref[...])


def _main(te, nv, buf, w1, b1, fw, fb, ew1g, eb1, ew2, eb2, ew3, eb3,
          dw1, db1, dw2, db2, dw3, db3):
    def tile_idx(t, te, nv):
        return (jnp.minimum(t, NT - 1), 0)

    def exp_idx3(t, te, nv):
        return (te[0, jnp.minimum(t, NT - 1)], 0, 0)

    def dec_idx(t, te, nv):
        return (0, jnp.maximum(t - NT, 0))

    const2 = lambda t, te, nv: (0, 0)
    grid_spec = pltpu.PrefetchScalarGridSpec(
        num_scalar_prefetch=2,
        grid=(NT + NDEC,),
        in_specs=[
            pl.BlockSpec((T, WORDS), tile_idx),
            pl.BlockSpec((1, SHAPE, 256), exp_idx3),
            pl.BlockSpec((1, 1, 256), exp_idx3),
            pl.BlockSpec((1, 256, 256), exp_idx3),
            pl.BlockSpec((1, 1, 256), exp_idx3),
            pl.BlockSpec((8, 256), const2),
            pl.BlockSpec((1, 256), const2),
            pl.BlockSpec((256, 512), const2),
            pl.BlockSpec((1, 512), const2),
            pl.BlockSpec((512, LAT), const2),
            pl.BlockSpec((1, LAT), const2),
            pl.BlockSpec((LAT, LAT), const2),
            pl.BlockSpec((1, LAT), const2),
            pl.BlockSpec((LAT, LAT), const2),
            pl.BlockSpec((1, LAT), const2),
            pl.BlockSpec((LAT, DTILE), dec_idx),
            pl.BlockSpec((1, DTILE), dec_idx),
        ],
        out_specs=[
            pl.BlockSpec((B, LAT), const2),
            pl.BlockSpec((B, DTILE), dec_idx),
        ],
        scratch_shapes=[pltpu.VMEM((B, LAT), BF16)],
    )
    return pl.pallas_call(
        _main_body,
        grid_spec=grid_spec,
        out_shape=[
            jax.ShapeDtypeStruct((B, LAT), F32),
            jax.ShapeDtypeStruct((B, DEC_OUT), F32),
        ],
    )(te, nv, buf, w1, b1, fw, fb, ew1g, eb1, ew2, eb2, ew3, eb3,
      dw1, db1, dw2, db2, dw3, db3)


def kernel(x, cats, se_W1, se_b1, se_W2, se_b2,
           enc_W1, enc_b1, enc_W2, enc_b2, enc_W3, enc_b3,
           dec_W1, dec_b1, dec_W2, dec_b2, dec_W3, dec_b3):
    cats32 = cats.astype(jnp.int32).reshape(128, 128)
    xp = jnp.transpose(x, (1, 0, 2))
    dest, te, nv, xt3, fw, fb, ew1g = _prep(
        cats32, xp, se_W2, se_b2.reshape(E, 1, SHAPE), enc_W1)
    buf = _sc_scatter(xt3.reshape(NTOT, WORDS), dest)

    latent, d = _main(
        te, nv, buf,
        se_W1.astype(BF16), se_b1.reshape(E, 1, 256),
        fw, fb,
        ew1g, enc_b1.reshape(1, 256),
        enc_W2.astype(BF16), enc_b2.reshape(1, 512),
        enc_W3.astype(BF16), enc_b3.reshape(1, LAT),
        dec_W1.astype(BF16), dec_b1.reshape(1, LAT),
        dec_W2.astype(BF16), dec_b2.reshape(1, LAT),
        dec_W3, dec_b3.reshape(1, DEC_OUT),
    )
    return d.reshape(B, NPTS, GEO), latent
